# baseline (speedup 1.0000x reference)
import jax
import jax.numpy as jnp
from jax import lax
from jax.experimental import pallas as pl
from jax.experimental.pallas import tpu as pltpu

T = 2048
D = 1024
E = 8
B = 256


def _gates_kernel(x_ref, w_ref, gates_ref, ent_ref):
    x = x_ref[...]
    w = w_ref[...]
    logits = lax.dot_general(x, w, (((1,), (1,)), ((), ())),
                             preferred_element_type=jnp.float32)
    m = jnp.max(logits, axis=1, keepdims=True)
    ex = jnp.exp(logits - m)
    g = ex / jnp.sum(ex, axis=1, keepdims=True)
    gates_ref[...] = g
    ent = -jnp.sum(g * jnp.log(g + 1e-9), axis=1)
    ent_ref[0, 0] = jnp.sum(ent) / jnp.float32(T)


def _diag_kernel(gates_ref, comb_ref, mask_ref):
    i = pl.program_id(0)
    j = pl.program_id(1)

    @pl.when(i != j)
    def _off_diag():
        comb_ref[...] = jnp.zeros_like(comb_ref)
        mask_ref[...] = jnp.zeros_like(mask_ref)

    @pl.when(i == j)
    def _on_diag():
        g = gates_ref[...]
        row = lax.broadcasted_iota(jnp.int32, (B, E, B), 0)
        col = lax.broadcasted_iota(jnp.int32, (B, E, B), 2)
        d = row == col
        gb = g[:, :, None]
        comb_ref[...] = jnp.where(d, gb, 0.0)
        mask_ref[...] = jnp.logical_and(d, gb != 0.0)


def kernel(input, W):
    gates, ent = pl.pallas_call(
        _gates_kernel,
        out_shape=(
            jax.ShapeDtypeStruct((T, E), jnp.float32),
            jax.ShapeDtypeStruct((1, 1), jnp.float32),
        ),
        out_specs=(
            pl.BlockSpec(memory_space=pltpu.VMEM),
            pl.BlockSpec(memory_space=pltpu.SMEM),
        ),
    )(input, W)

    nb = T // B
    comb, mask = pl.pallas_call(
        _diag_kernel,
        grid=(nb, nb),
        in_specs=(pl.BlockSpec((B, E), lambda i, j: (i, 0)),),
        out_specs=(
            pl.BlockSpec((B, E, B), lambda i, j: (i, 0, j)),
            pl.BlockSpec((B, E, B), lambda i, j: (i, 0, j)),
        ),
        out_shape=(
            jax.ShapeDtypeStruct((T, E, T), jnp.float32),
            jax.ShapeDtypeStruct((T, E, T), jnp.bool_),
        ),
    )(gates)

    l_aux = jnp.zeros((1,), dtype=jnp.float32)
    return (l_aux, comb, mask, ent[0, 0])


# speedup vs baseline: 1.0041x; 1.0041x over previous
import jax
import jax.numpy as jnp
from jax import lax
from jax.experimental import pallas as pl
from jax.experimental.pallas import tpu as pltpu

T = 2048
D = 1024
E = 8
B = 128
NB = T // B


def _gates_kernel(x_ref, w_ref, gates_ref, ent_ref):
    x = x_ref[...]
    w = w_ref[...]
    logits = lax.dot_general(x, w, (((1,), (1,)), ((), ())),
                             preferred_element_type=jnp.float32)
    m = jnp.max(logits, axis=1, keepdims=True)
    ex = jnp.exp(logits - m)
    g = ex / jnp.sum(ex, axis=1, keepdims=True)
    gates_ref[...] = g
    ent = -jnp.sum(g * jnp.log(g + 1e-9), axis=1)
    ent_ref[0, 0] = jnp.sum(ent) / jnp.float32(T)


def _diag_kernel(gates_ref, comb_ref, mask_ref, cbuf, csem):
    i = pl.program_id(0)
    b = lax.rem(i, 2)

    def comb_copy(buf_idx, step):
        return pltpu.make_async_copy(
            cbuf.at[buf_idx],
            comb_ref.at[pl.ds(step * B, B)],
            csem.at[buf_idx],
        )

    @pl.when(i >= 2)
    def _reclaim():
        comb_copy(b, i - 2).wait()
        cbuf[b, :, :, pl.ds((i - 2) * B, B)] = jnp.zeros(
            (B, E, B), jnp.float32)

    @pl.when(i < 2)
    def _init():
        cbuf[b] = jnp.zeros((B, E, T), jnp.float32)

    g = gates_ref[pl.ds(i * B, B), :]
    row = lax.broadcasted_iota(jnp.int32, (B, E, B), 0)
    col = lax.broadcasted_iota(jnp.int32, (B, E, B), 2)
    d = row == col
    gb = g[:, :, None]
    cbuf[b, :, :, pl.ds(i * B, B)] = jnp.where(d, gb, 0.0)
    comb_copy(b, i).start()

    mask_ref[...] = jnp.zeros((B, E, T), jnp.bool_)
    mask_ref[:, :, pl.ds(i * B, B)] = jnp.logical_and(d, gb != 0.0)

    @pl.when(i == NB - 1)
    def _drain():
        comb_copy(1 - b, NB - 2).wait()
        comb_copy(b, NB - 1).wait()


def kernel(input, W):
    gates, ent = pl.pallas_call(
        _gates_kernel,
        out_shape=(
            jax.ShapeDtypeStruct((T, E), jnp.float32),
            jax.ShapeDtypeStruct((1, 1), jnp.float32),
        ),
        out_specs=(
            pl.BlockSpec(memory_space=pltpu.VMEM),
            pl.BlockSpec(memory_space=pltpu.SMEM),
        ),
    )(input, W)

    comb, mask = pl.pallas_call(
        _diag_kernel,
        grid=(NB,),
        in_specs=(pl.BlockSpec(memory_space=pltpu.VMEM),),
        out_specs=(
            pl.BlockSpec(memory_space=pl.ANY),
            pl.BlockSpec((B, E, T), lambda i: (i, 0, 0)),
        ),
        out_shape=(
            jax.ShapeDtypeStruct((T, E, T), jnp.float32),
            jax.ShapeDtypeStruct((T, E, T), jnp.bool_),
        ),
        scratch_shapes=[
            pltpu.VMEM((2, B, E, T), jnp.float32),
            pltpu.SemaphoreType.DMA((2,)),
        ],
    )(gates)

    l_aux = jnp.zeros((1,), dtype=jnp.float32)
    return (l_aux, comb, mask, ent[0, 0])


# speedup vs baseline: 1.0054x; 1.0013x over previous
import jax
import jax.numpy as jnp
from jax import lax
from jax.experimental import pallas as pl
from jax.experimental.pallas import tpu as pltpu

T = 2048
D = 1024
E = 8
B = 128
NB = T // B


def _gates_kernel(x_ref, w_ref, gates_ref, ent_ref):
    x = x_ref[...]
    w = w_ref[...]
    logits = lax.dot_general(x, w, (((1,), (1,)), ((), ())),
                             preferred_element_type=jnp.float32)
    m = jnp.max(logits, axis=1, keepdims=True)
    ex = jnp.exp(logits - m)
    g = ex / jnp.sum(ex, axis=1, keepdims=True)
    gates_ref[...] = g
    ent = -jnp.sum(g * jnp.log(g + 1e-9), axis=1)
    ent_ref[0, 0] = jnp.sum(ent) / jnp.float32(T)


K = 8
BK = B // K


def _diag_kernel(gates_ref, comb_ref, mask_ref, cbuf, csem):
    i = pl.program_id(0)
    b = lax.rem(i, 2)

    def sub_copy(buf_idx, step, k):
        return pltpu.make_async_copy(
            cbuf.at[buf_idx, pl.ds(k * BK, BK)],
            comb_ref.at[pl.ds(step * B + k * BK, BK)],
            csem.at[buf_idx],
        )

    def start_all(buf_idx, step):
        for k in range(K):
            sub_copy(buf_idx, step, k).start()

    def wait_all(buf_idx, step):
        for k in range(K):
            sub_copy(buf_idx, step, k).wait()

    @pl.when(i >= 2)
    def _reclaim():
        wait_all(b, i - 2)
        cbuf[b, :, :, pl.ds((i - 2) * B, B)] = jnp.zeros(
            (B, E, B), jnp.float32)

    @pl.when(i < 2)
    def _init():
        cbuf[b] = jnp.zeros((B, E, T), jnp.float32)

    g = gates_ref[pl.ds(i * B, B), :]
    row = lax.broadcasted_iota(jnp.int32, (B, E, B), 0)
    col = lax.broadcasted_iota(jnp.int32, (B, E, B), 2)
    d = row == col
    gb = g[:, :, None]
    cbuf[b, :, :, pl.ds(i * B, B)] = jnp.where(d, gb, 0.0)
    start_all(b, i)

    mask_ref[...] = jnp.zeros((B, E, T), jnp.bool_)
    mask_ref[:, :, pl.ds(i * B, B)] = jnp.logical_and(d, gb != 0.0)

    @pl.when(i == NB - 1)
    def _drain():
        wait_all(1 - b, NB - 2)
        wait_all(b, NB - 1)


def kernel(input, W):
    gates, ent = pl.pallas_call(
        _gates_kernel,
        out_shape=(
            jax.ShapeDtypeStruct((T, E), jnp.float32),
            jax.ShapeDtypeStruct((1, 1), jnp.float32),
        ),
        out_specs=(
            pl.BlockSpec(memory_space=pltpu.VMEM),
            pl.BlockSpec(memory_space=pltpu.SMEM),
        ),
    )(input, W)

    comb, mask = pl.pallas_call(
        _diag_kernel,
        grid=(NB,),
        in_specs=(pl.BlockSpec(memory_space=pltpu.VMEM),),
        out_specs=(
            pl.BlockSpec(memory_space=pl.ANY),
            pl.BlockSpec((B, E, T), lambda i: (i, 0, 0)),
        ),
        out_shape=(
            jax.ShapeDtypeStruct((T, E, T), jnp.float32),
            jax.ShapeDtypeStruct((T, E, T), jnp.bool_),
        ),
        scratch_shapes=[
            pltpu.VMEM((2, B, E, T), jnp.float32),
            pltpu.SemaphoreType.DMA((2,)),
        ],
    )(gates)

    l_aux = jnp.zeros((1,), dtype=jnp.float32)
    return (l_aux, comb, mask, ent[0, 0])
